# baseline (speedup 1.0000x reference)
import functools

import jax
import jax.numpy as jnp
from jax import lax
from jax.experimental import pallas as pl
from jax.experimental.pallas import tpu as pltpu
from jax.experimental.pallas import tpu_sc as plsc

N = 10000
E = 320000
D = 128
R = 4
NW = 32
P = R // 2
ETILES = E // 128
WTILES = ETILES // NW + 1
WEDGE = WTILES * 128

_mesh = plsc.VectorSubcoreMesh(core_axis_name="c", subcore_axis_name="s")


@functools.partial(
    pl.kernel,
    out_type=jax.ShapeDtypeStruct((NW * P, N), jnp.int32),
    mesh=_mesh,
    compiler_params=pltpu.CompilerParams(needs_layout_passes=False),
    scratch_types=[
        pltpu.VMEM((P, N), jnp.int32),
        pltpu.VMEM((2, WEDGE), jnp.int32),
        pltpu.VMEM((WEDGE,), jnp.int32),
        pltpu.SemaphoreType.DMA,
        pltpu.SemaphoreType.DMA,
    ],
)
def _sc_histogram(ei_hbm, et_hbm, out_hbm, hist_v, ei_v, et_v, sem1, sem2):
    wid = lax.axis_index("s") * 2 + lax.axis_index("c")
    t0 = (wid * ETILES) // NW
    t1 = ((wid + 1) * ETILES) // NW
    start = t0
    n_it = t1 - t0

    cp1 = pltpu.async_copy(
        ei_hbm.at[:, pl.ds(start * 128, WEDGE)], ei_v, sem1)
    cp2 = pltpu.async_copy(
        et_hbm.at[pl.ds(start * 128, WEDGE)], et_v, sem2)

    zeros = jnp.zeros((16,), jnp.int32)

    @plsc.parallel_loop(0, N, step=16 * 25)
    def zero_body(base):
        for j in range(25):
            off = base + j * 16
            hist_v[0, pl.ds(off, 16)] = zeros
            hist_v[1, pl.ds(off, 16)] = zeros

    one = jnp.ones((16,), jnp.int32)
    hi_one = jnp.full((16,), 1 << 16, jnp.int32)
    two = jnp.full((16,), 2, jnp.int32)

    cp1.wait()
    cp2.wait()

    @plsc.parallel_loop(0, n_it * 128, step=128)
    def edge_body(base):
        for j in range(8):
            off = base + j * 16
            s = ei_v[0, pl.ds(off, 16)]
            t = et_v[pl.ds(off, 16)]
            val = jnp.where(t >= two, hi_one, one)
            plsc.addupdate_scatter(hist_v, [t & one, s], val)

    pltpu.sync_copy(hist_v, out_hbm.at[pl.ds(wid * P, P)])


def _tc_body(c_ref, x_ref, w_ref, wl_ref, b_ref, o_ref):
    c = c_ref[...]
    low = (c & 0xFFFF).astype(jnp.float32)
    high = (c >> 16).astype(jnp.float32)
    g = lax.broadcasted_iota(jnp.int32, (P, NW * P), 1)
    p = lax.broadcasted_iota(jnp.int32, (P, NW * P), 0)
    sel = (g % P == p).astype(jnp.float32)
    c01 = jnp.dot(sel, low, preferred_element_type=jnp.float32)
    c23 = jnp.dot(sel, high, preferred_element_type=jnp.float32)
    c4 = jnp.concatenate([c01, c23], axis=0)
    x = x_ref[...]
    s4 = jnp.dot(c4, x, preferred_element_type=jnp.float32)
    colsum = jnp.sum(x, axis=0, keepdims=True)
    out = jnp.dot(colsum, wl_ref[...], preferred_element_type=jnp.float32)
    for rr in range(R):
        out = out + jnp.dot(s4[rr:rr + 1, :], w_ref[rr],
                            preferred_element_type=jnp.float32)
    o_ref[...] = out * (1.0 / N) + b_ref[...]


_tc_final = pl.pallas_call(
    _tc_body,
    out_shape=jax.ShapeDtypeStruct((1, D), jnp.float32),
)


@jax.jit
def kernel(x, edge_index, edge_type, W, W_loop, bias):
    counts = _sc_histogram(edge_index, edge_type)
    return _tc_final(counts, x, W, W_loop, bias.reshape(1, D))
